# baseline (speedup 1.0000x reference)
import functools

import jax
import jax.numpy as jnp
from jax import lax
from jax.experimental import pallas as pl
from jax.experimental.pallas import tpu as pltpu
from jax.experimental.pallas import tpu_sc as plsc

N = 10000
E = 320000
D = 128
NRBF = 16
NPAD = 10240

NC = 2
NS = 16
NW = NC * NS
EPW = E // NW
CH = 80
NCHUNK = EPW // CH
LANES = 16
TBL_SL = NPAD // NS
ECORE = E // NS
CHG = 80
IDXW = 80
NCHG = ECORE // CHG
NBUF = 2

BN = 1000
BE = 2000


def _mesh():
    return plsc.VectorSubcoreMesh(core_axis_name="c", subcore_axis_name="s")


def _tables_body(h_ref, wa_ref, wb_ref, a_ref, b_ref):
    h = h_ref[...]
    a_ref[...] = jnp.dot(h, wa_ref[...], preferred_element_type=jnp.float32)
    b_ref[...] = jnp.dot(h, wb_ref[...], preferred_element_type=jnp.float32)


def _make_tables(h, w1m_a, w1m_b):
    return pl.pallas_call(
        _tables_body,
        grid=(N // BN,),
        in_specs=[
            pl.BlockSpec((BN, D), lambda i: (i, 0)),
            pl.BlockSpec((D, D), lambda i: (0, 0)),
            pl.BlockSpec((D, D), lambda i: (0, 0)),
        ],
        out_specs=[
            pl.BlockSpec((BN, D), lambda i: (i, 0)),
            pl.BlockSpec((BN, D), lambda i: (i, 0)),
        ],
        out_shape=[
            jax.ShapeDtypeStruct((NPAD, D), jnp.float32),
            jax.ShapeDtypeStruct((NPAD, D), jnp.float32),
        ],
    )(h, w1m_a, w1m_b)


@functools.lru_cache(maxsize=None)
def _sc_gather_fn():
    @functools.partial(
        pl.kernel,
        mesh=_mesh(),
        compiler_params=pltpu.CompilerParams(needs_layout_passes=False),
        out_type=[
            jax.ShapeDtypeStruct((E, D), jnp.float32),
            jax.ShapeDtypeStruct((E, D), jnp.float32),
            jax.ShapeDtypeStruct((E,), jnp.float32),
        ],
        scratch_types=[
            pltpu.VMEM((CHG,), jnp.int32),
            pltpu.VMEM((CHG,), jnp.int32),
            pltpu.VMEM((CHG,), jnp.int32),
            pltpu.VMEM((CHG,), jnp.int32),
            pltpu.VMEM((CHG, D), jnp.float32),
            pltpu.VMEM((CHG, D), jnp.float32),
            pltpu.VMEM((CHG,), jnp.float32),
            pltpu.VMEM((CHG,), jnp.float32),
            pltpu.VMEM((N,), jnp.float32),
            pltpu.VMEM((N,), jnp.float32),
            pltpu.VMEM_SHARED((NPAD, D), jnp.float32),
            pltpu.SemaphoreType.DMA,
            pltpu.SemaphoreType.DMA,
            pltpu.SemaphoreType.DMA,
            pltpu.SemaphoreType.DMA,
        ],
    )
    def _sc_gather(a_hbm, b_hbm, src_hbm, dst_hbm, x0_hbm, x1_hbm,
                   oa_hbm, ob_hbm, r2_hbm,
                   idxa0, idxa1, idxb0, idxb1, rows0, rows1, r2a0, r2a1,
                   x0_v, x1_v, tbl_sh, semi, semg, semw, semw2):
        cid = lax.axis_index("c")
        sid = lax.axis_index("s")
        tsl = pl.ds(sid * TBL_SL, TBL_SL)

        @pl.when(cid == 0)
        def _stage_a():
            pltpu.sync_copy(a_hbm.at[tsl], tbl_sh.at[tsl])
            pltpu.sync_copy(x0_hbm, x0_v)
            pltpu.sync_copy(x1_hbm, x1_v)

        @pl.when(cid == 1)
        def _stage_b():
            pltpu.sync_copy(b_hbm.at[tsl], tbl_sh.at[tsl])

        plsc.subcore_barrier()

        ebase = sid * ECORE
        idxa = (idxa0, idxa1)
        idxb = (idxb0, idxb1)
        rows = (rows0, rows1)
        r2b = (r2a0, r2a1)

        def _issue_idx0(g, b):
            pltpu.async_copy(src_hbm.at[pl.ds(ebase + g * CHG, CHG)],
                             idxa[b], semi)
            pltpu.async_copy(dst_hbm.at[pl.ds(ebase + g * CHG, CHG)],
                             idxb[b], semi)

        def _drain_idx0(b):
            pltpu.make_async_copy(src_hbm.at[pl.ds(0, CHG)],
                                  idxa[b], semi).wait()
            pltpu.make_async_copy(dst_hbm.at[pl.ds(0, CHG)],
                                  idxb[b], semi).wait()

        def _chunk0(g, t, b):
            base = ebase + g * CHG
            _drain_idx0(b)

            @pl.when(g + 1 < NCHG)
            def _prefetch():
                _issue_idx0(g + 1, 1 - b)

            @pl.when(t >= 1)
            def _drain_writes():
                pltpu.make_async_copy(rows[b], oa_hbm.at[pl.ds(base, CHG)],
                                      semw).wait()
                pltpu.make_async_copy(r2b[b], r2_hbm.at[pl.ds(base, CHG)],
                                      semw2).wait()

            gth = pltpu.async_copy(tbl_sh.at[idxa[b]], rows[b], semg)
            for j in range(CHG // LANES):
                ivs = idxa[b][pl.ds(j * LANES, LANES)]
                ivd = idxb[b][pl.ds(j * LANES, LANES)]
                d0 = (plsc.load_gather(x0_v, [ivs])
                      - plsc.load_gather(x0_v, [ivd]))
                d1 = (plsc.load_gather(x1_v, [ivs])
                      - plsc.load_gather(x1_v, [ivd]))
                r2b[b][pl.ds(j * LANES, LANES)] = d0 * d0 + d1 * d1
            gth.wait()
            pltpu.async_copy(rows[b], oa_hbm.at[pl.ds(base, CHG)], semw)
            pltpu.async_copy(r2b[b], r2_hbm.at[pl.ds(base, CHG)], semw2)

        def _issue_idx1(g, b):
            pltpu.async_copy(dst_hbm.at[pl.ds(ebase + g * CHG, CHG)],
                             idxb[b], semi)

        def _chunk1(g, t, b):
            base = ebase + g * CHG
            pltpu.make_async_copy(dst_hbm.at[pl.ds(0, CHG)],
                                  idxb[b], semi).wait()

            @pl.when(g + 1 < NCHG)
            def _prefetch():
                _issue_idx1(g + 1, 1 - b)

            @pl.when(t >= 1)
            def _drain_writes():
                pltpu.make_async_copy(rows[b], ob_hbm.at[pl.ds(base, CHG)],
                                      semw).wait()

            gth = pltpu.async_copy(tbl_sh.at[idxb[b]], rows[b], semg)
            gth.wait()
            pltpu.async_copy(rows[b], ob_hbm.at[pl.ds(base, CHG)], semw)

        @pl.when(cid == 0)
        def _run0():
            _issue_idx0(0, 0)

            def pair(t, carry):
                _chunk0(2 * t, t, 0)
                _chunk0(2 * t + 1, t, 1)
                return carry

            lax.fori_loop(0, NCHG // 2, pair, 0)
            for b in range(NBUF):
                pltpu.make_async_copy(rows[b], oa_hbm.at[pl.ds(0, CHG)],
                                      semw).wait()
                pltpu.make_async_copy(r2b[b], r2_hbm.at[pl.ds(0, CHG)],
                                      semw2).wait()

        @pl.when(cid == 1)
        def _run1():
            _issue_idx1(0, 0)

            def pair(t, carry):
                _chunk1(2 * t, t, 0)
                _chunk1(2 * t + 1, t, 1)
                return carry

            lax.fori_loop(0, NCHG // 2, pair, 0)
            for b in range(NBUF):
                pltpu.make_async_copy(rows[b], ob_hbm.at[pl.ds(0, CHG)],
                                      semw).wait()

    return _sc_gather


def _edge_body(za_ref, zb_ref, r2_ref, wr_ref, b1_ref, w2_ref, b2_ref,
               cen_ref, gam_ref, out_ref):
    pre = za_ref[...] + zb_ref[...]
    r = jnp.sqrt(r2_ref[...] + 1e-8)
    diff = r - cen_ref[...]
    rbf = jnp.exp(-gam_ref[...] * diff * diff)
    z = pre + jnp.dot(rbf, wr_ref[...],
                      preferred_element_type=jnp.float32) + b1_ref[...]
    s = z * jax.nn.sigmoid(z)
    out_ref[...] = jnp.dot(s, w2_ref[...],
                           preferred_element_type=jnp.float32) + b2_ref[...]


def _make_edges(oa, ob, r2, w1m_r, b1m, w2m, b2m, cen_row, gam_row):
    return pl.pallas_call(
        _edge_body,
        grid=(E // BE,),
        in_specs=[
            pl.BlockSpec((BE, D), lambda i: (i, 0)),
            pl.BlockSpec((BE, D), lambda i: (i, 0)),
            pl.BlockSpec((BE, 1), lambda i: (i, 0)),
            pl.BlockSpec((NRBF, D), lambda i: (0, 0)),
            pl.BlockSpec((1, D), lambda i: (0, 0)),
            pl.BlockSpec((D, D), lambda i: (0, 0)),
            pl.BlockSpec((1, D), lambda i: (0, 0)),
            pl.BlockSpec((1, NRBF), lambda i: (0, 0)),
            pl.BlockSpec((1, NRBF), lambda i: (0, 0)),
        ],
        out_specs=pl.BlockSpec((BE, D), lambda i: (i, 0)),
        out_shape=jax.ShapeDtypeStruct((E, D), jnp.float32),
    )(oa, ob, r2, w1m_r, b1m, w2m, b2m, cen_row, gam_row)


@functools.lru_cache(maxsize=None)
def _sc_scatter_fn():
    @functools.partial(
        pl.kernel,
        mesh=_mesh(),
        out_type=jax.ShapeDtypeStruct((NC, NPAD, D), jnp.float32),
        scratch_types=[
            pltpu.VMEM((CH,), jnp.int32),
            pltpu.VMEM((CH,), jnp.int32),
            pltpu.VMEM((CH, D), jnp.float32),
            pltpu.VMEM((CH, D), jnp.float32),
            pltpu.VMEM_SHARED((NPAD, D), jnp.float32),
            pltpu.SemaphoreType.DMA,
            pltpu.SemaphoreType.DMA,
            pltpu.SemaphoreType.DMA,
        ],
    )
    def _sc_scatter(m_hbm, dst_hbm, out_hbm, idx0, idx1, rows0, rows1,
                    acc_sh, semi, semm, sema):
        cid = lax.axis_index("c")
        sid = lax.axis_index("s")
        wid = sid * NC + cid
        rows_per_sub = NPAD // NS
        sl = pl.ds(sid * rows_per_sub, rows_per_sub)
        idxb = (idx0, idx1)
        rows = (rows0, rows1)

        def zrow(i, carry):
            for k in range(D // LANES):
                rows0[i, pl.ds(k * LANES, LANES)] = jnp.zeros(
                    (LANES,), jnp.float32)
            return carry

        lax.fori_loop(0, CH, zrow, 0)
        for k in range(rows_per_sub // CH):
            pltpu.sync_copy(
                rows0, acc_sh.at[pl.ds(sid * rows_per_sub + k * CH, CH)])
        plsc.subcore_barrier()

        ebase = wid * EPW
        idxb = (idx0, idx1)
        rows = (rows0, rows1)

        def _issue(g, b):
            pltpu.async_copy(dst_hbm.at[pl.ds(ebase + g * CH, CH)],
                             idxb[b], semi)
            pltpu.async_copy(m_hbm.at[pl.ds(ebase + g * CH, CH)],
                             rows[b], semm)

        def _chunk(g, b, drain_prev):
            nb = 1 - b

            @pl.when(drain_prev)
            def _drain_add():
                pltpu.make_async_copy(rows[nb], acc_sh.at[idxb[nb]],
                                      sema).wait()

            pltpu.make_async_copy(dst_hbm.at[pl.ds(0, CH)], idxb[b],
                                  semi).wait()
            pltpu.make_async_copy(m_hbm.at[pl.ds(0, CH)], rows[b],
                                  semm).wait()

            @pl.when(g + 1 < NCHUNK)
            def _prefetch():
                _issue(g + 1, nb)

            pltpu.async_copy(rows[b], acc_sh.at[idxb[b]], sema, add=True)

        _issue(0, 0)

        def pair(t, carry):
            _chunk(2 * t, 0, t >= 1)
            _chunk(2 * t + 1, 1, t >= 0)
            return carry

        lax.fori_loop(0, NCHUNK // 2, pair, 0)
        if NCHUNK % 2:
            _chunk(NCHUNK - 1, 0, NCHUNK >= 2)
            pltpu.make_async_copy(rows0, acc_sh.at[idx0], sema).wait()
        else:
            pltpu.make_async_copy(rows1, acc_sh.at[idx1], sema).wait()

        plsc.subcore_barrier()
        pltpu.sync_copy(acc_sh.at[sl], out_hbm.at[cid, sl])

    return _sc_scatter


def _node_body(h_ref, s0_ref, s1_ref, w1ha_ref, w1hb_ref,
               b1h_ref, w2h_ref, b2h_ref, lnw_ref, lnb_ref, out_ref):
    h = h_ref[...]
    agg = s0_ref[...] + s1_ref[...]
    pre = (jnp.dot(h, w1ha_ref[...], preferred_element_type=jnp.float32)
           + jnp.dot(agg, w1hb_ref[...], preferred_element_type=jnp.float32)
           + b1h_ref[...])
    t = pre * jax.nn.sigmoid(pre)
    h_up = jnp.dot(t, w2h_ref[...],
                   preferred_element_type=jnp.float32) + b2h_ref[...]
    y = h + h_up
    mu = jnp.mean(y, axis=1, keepdims=True)
    var = jnp.mean((y - mu) ** 2, axis=1, keepdims=True)
    out_ref[...] = ((y - mu) * lax.rsqrt(var + 1e-5) * lnw_ref[...]
                    + lnb_ref[...])


def _make_nodes(h, s0, s1, w1h_a, w1h_b, b1h, w2h, b2h, lnw, lnb):
    return pl.pallas_call(
        _node_body,
        grid=(N // BN,),
        in_specs=[
            pl.BlockSpec((BN, D), lambda i: (i, 0)),
            pl.BlockSpec((BN, D), lambda i: (i, 0)),
            pl.BlockSpec((BN, D), lambda i: (i, 0)),
            pl.BlockSpec((D, D), lambda i: (0, 0)),
            pl.BlockSpec((D, D), lambda i: (0, 0)),
            pl.BlockSpec((1, D), lambda i: (0, 0)),
            pl.BlockSpec((D, D), lambda i: (0, 0)),
            pl.BlockSpec((1, D), lambda i: (0, 0)),
            pl.BlockSpec((1, D), lambda i: (0, 0)),
            pl.BlockSpec((1, D), lambda i: (0, 0)),
        ],
        out_specs=pl.BlockSpec((BN, D), lambda i: (i, 0)),
        out_shape=jax.ShapeDtypeStruct((N, D), jnp.float32),
    )(h, s0, s1, w1h_a, w1h_b, b1h, w2h, b2h, lnw, lnb)


def kernel(h, x, edge_index, W1m, b1m, W2m, b2m, W1h, b1h, W2h, b2h,
           ln_w, ln_b, centers, gamma):
    src = edge_index[0].astype(jnp.int32)
    dst = edge_index[1].astype(jnp.int32)
    x0 = x[:, 0]
    x1 = x[:, 1]
    cen_row = centers[None, :]
    gam_row = jnp.full((1, NRBF), gamma, jnp.float32)

    a_tab, b_tab = _make_tables(h, W1m[:D], W1m[D:2 * D])
    oa, ob, r2 = _sc_gather_fn()(a_tab, b_tab, src, dst, x0, x1)
    m = _make_edges(oa, ob, r2[:, None], W1m[2 * D:], b1m[None, :],
                    W2m, b2m[None, :], cen_row, gam_row)
    s_part = _sc_scatter_fn()(m, dst)
    out = _make_nodes(h, s_part[0, :N], s_part[1, :N], W1h[:D], W1h[D:],
                      b1h[None, :], W2h, b2h[None, :], ln_w[None, :],
                      ln_b[None, :])
    return out


# speedup vs baseline: 1.1241x; 1.1241x over previous
import functools

import jax
import jax.numpy as jnp
from jax import lax
from jax.experimental import pallas as pl
from jax.experimental.pallas import tpu as pltpu
from jax.experimental.pallas import tpu_sc as plsc

N = 10000
E = 320000
D = 128
NRBF = 16
NPAD = 10240

NC = 2
NS = 16
NW = NC * NS
EPW = E // NW
CH = 80
NCHUNK = EPW // CH
LANES = 16
TBL_SL = NPAD // NS
ECORE = E // NS
CHG = 80
IDXW = 80
NCHG = ECORE // CHG
NBUF = 2

BN = 1000
BE = 2560


def _mesh():
    return plsc.VectorSubcoreMesh(core_axis_name="c", subcore_axis_name="s")


def _tables_body(h_ref, wa_ref, wb_ref, a_ref, b_ref):
    h = h_ref[...]
    a_ref[...] = jnp.dot(h, wa_ref[...], preferred_element_type=jnp.float32)
    b_ref[...] = jnp.dot(h, wb_ref[...], preferred_element_type=jnp.float32)


def _make_tables(h, w1m_a, w1m_b):
    return pl.pallas_call(
        _tables_body,
        grid=(N // BN,),
        in_specs=[
            pl.BlockSpec((BN, D), lambda i: (i, 0)),
            pl.BlockSpec((D, D), lambda i: (0, 0)),
            pl.BlockSpec((D, D), lambda i: (0, 0)),
        ],
        out_specs=[
            pl.BlockSpec((BN, D), lambda i: (i, 0)),
            pl.BlockSpec((BN, D), lambda i: (i, 0)),
        ],
        out_shape=[
            jax.ShapeDtypeStruct((NPAD, D), jnp.float32),
            jax.ShapeDtypeStruct((NPAD, D), jnp.float32),
        ],
    )(h, w1m_a, w1m_b)


@functools.lru_cache(maxsize=None)
def _sc_gather_fn():
    @functools.partial(
        pl.kernel,
        mesh=_mesh(),
        compiler_params=pltpu.CompilerParams(needs_layout_passes=False),
        out_type=[
            jax.ShapeDtypeStruct((E, D), jnp.float32),
            jax.ShapeDtypeStruct((E, D), jnp.float32),
            jax.ShapeDtypeStruct((E,), jnp.float32),
        ],
        scratch_types=[
            pltpu.VMEM((CHG,), jnp.int32),
            pltpu.VMEM((CHG,), jnp.int32),
            pltpu.VMEM((CHG,), jnp.int32),
            pltpu.VMEM((CHG,), jnp.int32),
            pltpu.VMEM((CHG, D), jnp.float32),
            pltpu.VMEM((CHG, D), jnp.float32),
            pltpu.VMEM((CHG,), jnp.float32),
            pltpu.VMEM((CHG,), jnp.float32),
            pltpu.VMEM((N,), jnp.float32),
            pltpu.VMEM((N,), jnp.float32),
            pltpu.VMEM_SHARED((NPAD, D), jnp.float32),
            pltpu.SemaphoreType.DMA,
            pltpu.SemaphoreType.DMA,
            pltpu.SemaphoreType.DMA,
            pltpu.SemaphoreType.DMA,
        ],
    )
    def _sc_gather(a_hbm, b_hbm, src_hbm, dst_hbm, x0_hbm, x1_hbm,
                   oa_hbm, ob_hbm, r2_hbm,
                   idxa0, idxa1, idxb0, idxb1, rows0, rows1, r2a0, r2a1,
                   x0_v, x1_v, tbl_sh, semi, semg, semw, semw2):
        cid = lax.axis_index("c")
        sid = lax.axis_index("s")
        tsl = pl.ds(sid * TBL_SL, TBL_SL)

        @pl.when(cid == 0)
        def _stage_a():
            pltpu.sync_copy(a_hbm.at[tsl], tbl_sh.at[tsl])
            pltpu.sync_copy(x0_hbm, x0_v)
            pltpu.sync_copy(x1_hbm, x1_v)

        @pl.when(cid == 1)
        def _stage_b():
            pltpu.sync_copy(b_hbm.at[tsl], tbl_sh.at[tsl])

        plsc.subcore_barrier()

        ebase = sid * ECORE
        idxa = (idxa0, idxa1)
        idxb = (idxb0, idxb1)
        rows = (rows0, rows1)
        r2b = (r2a0, r2a1)

        def _issue_idx0(g, b):
            pltpu.async_copy(src_hbm.at[pl.ds(ebase + g * CHG, CHG)],
                             idxa[b], semi)
            pltpu.async_copy(dst_hbm.at[pl.ds(ebase + g * CHG, CHG)],
                             idxb[b], semi)

        def _drain_idx0(b):
            pltpu.make_async_copy(src_hbm.at[pl.ds(0, CHG)],
                                  idxa[b], semi).wait()
            pltpu.make_async_copy(dst_hbm.at[pl.ds(0, CHG)],
                                  idxb[b], semi).wait()

        def _chunk0(g, t, b):
            base = ebase + g * CHG
            _drain_idx0(b)

            @pl.when(g + 1 < NCHG)
            def _prefetch():
                _issue_idx0(g + 1, 1 - b)

            @pl.when(t >= 1)
            def _drain_writes():
                pltpu.make_async_copy(rows[b], oa_hbm.at[pl.ds(base, CHG)],
                                      semw).wait()
                pltpu.make_async_copy(r2b[b], r2_hbm.at[pl.ds(base, CHG)],
                                      semw2).wait()

            gth = pltpu.async_copy(tbl_sh.at[idxa[b]], rows[b], semg)
            for j in range(CHG // LANES):
                ivs = idxa[b][pl.ds(j * LANES, LANES)]
                ivd = idxb[b][pl.ds(j * LANES, LANES)]
                d0 = (plsc.load_gather(x0_v, [ivs])
                      - plsc.load_gather(x0_v, [ivd]))
                d1 = (plsc.load_gather(x1_v, [ivs])
                      - plsc.load_gather(x1_v, [ivd]))
                r2b[b][pl.ds(j * LANES, LANES)] = d0 * d0 + d1 * d1
            gth.wait()
            pltpu.async_copy(rows[b], oa_hbm.at[pl.ds(base, CHG)], semw)
            pltpu.async_copy(r2b[b], r2_hbm.at[pl.ds(base, CHG)], semw2)

        def _issue_idx1(g, b):
            pltpu.async_copy(dst_hbm.at[pl.ds(ebase + g * CHG, CHG)],
                             idxb[b], semi)

        def _chunk1(g, t, b):
            base = ebase + g * CHG
            pltpu.make_async_copy(dst_hbm.at[pl.ds(0, CHG)],
                                  idxb[b], semi).wait()

            @pl.when(g + 1 < NCHG)
            def _prefetch():
                _issue_idx1(g + 1, 1 - b)

            @pl.when(t >= 1)
            def _drain_writes():
                pltpu.make_async_copy(rows[b], ob_hbm.at[pl.ds(base, CHG)],
                                      semw).wait()

            gth = pltpu.async_copy(tbl_sh.at[idxb[b]], rows[b], semg)
            gth.wait()
            pltpu.async_copy(rows[b], ob_hbm.at[pl.ds(base, CHG)], semw)

        @pl.when(cid == 0)
        def _run0():
            _issue_idx0(0, 0)

            def pair(t, carry):
                _chunk0(2 * t, t, 0)
                _chunk0(2 * t + 1, t, 1)
                return carry

            lax.fori_loop(0, NCHG // 2, pair, 0)
            for b in range(NBUF):
                pltpu.make_async_copy(rows[b], oa_hbm.at[pl.ds(0, CHG)],
                                      semw).wait()
                pltpu.make_async_copy(r2b[b], r2_hbm.at[pl.ds(0, CHG)],
                                      semw2).wait()

        @pl.when(cid == 1)
        def _run1():
            _issue_idx1(0, 0)

            def pair(t, carry):
                _chunk1(2 * t, t, 0)
                _chunk1(2 * t + 1, t, 1)
                return carry

            lax.fori_loop(0, NCHG // 2, pair, 0)
            for b in range(NBUF):
                pltpu.make_async_copy(rows[b], ob_hbm.at[pl.ds(0, CHG)],
                                      semw).wait()

    return _sc_gather


def _edge_body(za_ref, zb_ref, r2_ref, wr_ref, b1_ref, w2_ref, b2_ref,
               cen_ref, gam_ref, out_ref):
    pre = za_ref[...] + zb_ref[...]
    r = jnp.sqrt(r2_ref[...] + 1e-8)
    parts = []
    for s in range(8):
        diff = r[:, s:s + 1] - cen_ref[...]
        rbf = jnp.exp(-gam_ref[...] * diff * diff)
        parts.append(jnp.dot(rbf, wr_ref[...],
                             preferred_element_type=jnp.float32))
    zr = jnp.reshape(jnp.stack(parts, axis=1), (BE, D))
    z = pre + zr + b1_ref[...]
    s = z * jax.nn.sigmoid(z)
    out_ref[...] = jnp.dot(s, w2_ref[...],
                           preferred_element_type=jnp.float32) + b2_ref[...]


def _make_edges(oa, ob, r2, w1m_r, b1m, w2m, b2m, cen_row, gam_row):
    return pl.pallas_call(
        _edge_body,
        grid=(E // BE,),
        in_specs=[
            pl.BlockSpec((BE, D), lambda i: (i, 0)),
            pl.BlockSpec((BE, D), lambda i: (i, 0)),
            pl.BlockSpec((BE // 8, 8), lambda i: (i, 0)),
            pl.BlockSpec((NRBF, D), lambda i: (0, 0)),
            pl.BlockSpec((1, D), lambda i: (0, 0)),
            pl.BlockSpec((D, D), lambda i: (0, 0)),
            pl.BlockSpec((1, D), lambda i: (0, 0)),
            pl.BlockSpec((1, NRBF), lambda i: (0, 0)),
            pl.BlockSpec((1, NRBF), lambda i: (0, 0)),
        ],
        out_specs=pl.BlockSpec((BE, D), lambda i: (i, 0)),
        out_shape=jax.ShapeDtypeStruct((E, D), jnp.float32),
    )(oa, ob, r2, w1m_r, b1m, w2m, b2m, cen_row, gam_row)


@functools.lru_cache(maxsize=None)
def _sc_scatter_fn():
    @functools.partial(
        pl.kernel,
        mesh=_mesh(),
        out_type=jax.ShapeDtypeStruct((NC, NPAD, D), jnp.float32),
        scratch_types=[
            pltpu.VMEM((CH,), jnp.int32),
            pltpu.VMEM((CH,), jnp.int32),
            pltpu.VMEM((CH, D), jnp.float32),
            pltpu.VMEM((CH, D), jnp.float32),
            pltpu.VMEM_SHARED((NPAD, D), jnp.float32),
            pltpu.SemaphoreType.DMA,
            pltpu.SemaphoreType.DMA,
            pltpu.SemaphoreType.DMA,
        ],
    )
    def _sc_scatter(m_hbm, dst_hbm, out_hbm, idx0, idx1, rows0, rows1,
                    acc_sh, semi, semm, sema):
        cid = lax.axis_index("c")
        sid = lax.axis_index("s")
        wid = sid * NC + cid
        rows_per_sub = NPAD // NS
        sl = pl.ds(sid * rows_per_sub, rows_per_sub)
        idxb = (idx0, idx1)
        rows = (rows0, rows1)

        def zrow(i, carry):
            for k in range(D // LANES):
                rows0[i, pl.ds(k * LANES, LANES)] = jnp.zeros(
                    (LANES,), jnp.float32)
            return carry

        lax.fori_loop(0, CH, zrow, 0)
        for k in range(rows_per_sub // CH):
            pltpu.sync_copy(
                rows0, acc_sh.at[pl.ds(sid * rows_per_sub + k * CH, CH)])
        plsc.subcore_barrier()

        ebase = wid * EPW
        idxb = (idx0, idx1)
        rows = (rows0, rows1)

        def _issue(g, b):
            pltpu.async_copy(dst_hbm.at[pl.ds(ebase + g * CH, CH)],
                             idxb[b], semi)
            pltpu.async_copy(m_hbm.at[pl.ds(ebase + g * CH, CH)],
                             rows[b], semm)

        def _chunk(g, b, drain_prev):
            nb = 1 - b

            @pl.when(drain_prev)
            def _drain_add():
                pltpu.make_async_copy(rows[nb], acc_sh.at[idxb[nb]],
                                      sema).wait()

            pltpu.make_async_copy(dst_hbm.at[pl.ds(0, CH)], idxb[b],
                                  semi).wait()
            pltpu.make_async_copy(m_hbm.at[pl.ds(0, CH)], rows[b],
                                  semm).wait()

            @pl.when(g + 1 < NCHUNK)
            def _prefetch():
                _issue(g + 1, nb)

            pltpu.async_copy(rows[b], acc_sh.at[idxb[b]], sema, add=True)

        _issue(0, 0)

        def pair(t, carry):
            _chunk(2 * t, 0, t >= 1)
            _chunk(2 * t + 1, 1, t >= 0)
            return carry

        lax.fori_loop(0, NCHUNK // 2, pair, 0)
        if NCHUNK % 2:
            _chunk(NCHUNK - 1, 0, NCHUNK >= 2)
            pltpu.make_async_copy(rows0, acc_sh.at[idx0], sema).wait()
        else:
            pltpu.make_async_copy(rows1, acc_sh.at[idx1], sema).wait()

        plsc.subcore_barrier()
        pltpu.sync_copy(acc_sh.at[sl], out_hbm.at[cid, sl])

    return _sc_scatter


def _node_body(h_ref, s0_ref, s1_ref, w1ha_ref, w1hb_ref,
               b1h_ref, w2h_ref, b2h_ref, lnw_ref, lnb_ref, out_ref):
    h = h_ref[...]
    agg = s0_ref[...] + s1_ref[...]
    pre = (jnp.dot(h, w1ha_ref[...], preferred_element_type=jnp.float32)
           + jnp.dot(agg, w1hb_ref[...], preferred_element_type=jnp.float32)
           + b1h_ref[...])
    t = pre * jax.nn.sigmoid(pre)
    h_up = jnp.dot(t, w2h_ref[...],
                   preferred_element_type=jnp.float32) + b2h_ref[...]
    y = h + h_up
    mu = jnp.mean(y, axis=1, keepdims=True)
    var = jnp.mean((y - mu) ** 2, axis=1, keepdims=True)
    out_ref[...] = ((y - mu) * lax.rsqrt(var + 1e-5) * lnw_ref[...]
                    + lnb_ref[...])


def _make_nodes(h, s0, s1, w1h_a, w1h_b, b1h, w2h, b2h, lnw, lnb):
    return pl.pallas_call(
        _node_body,
        grid=(N // BN,),
        in_specs=[
            pl.BlockSpec((BN, D), lambda i: (i, 0)),
            pl.BlockSpec((BN, D), lambda i: (i, 0)),
            pl.BlockSpec((BN, D), lambda i: (i, 0)),
            pl.BlockSpec((D, D), lambda i: (0, 0)),
            pl.BlockSpec((D, D), lambda i: (0, 0)),
            pl.BlockSpec((1, D), lambda i: (0, 0)),
            pl.BlockSpec((D, D), lambda i: (0, 0)),
            pl.BlockSpec((1, D), lambda i: (0, 0)),
            pl.BlockSpec((1, D), lambda i: (0, 0)),
            pl.BlockSpec((1, D), lambda i: (0, 0)),
        ],
        out_specs=pl.BlockSpec((BN, D), lambda i: (i, 0)),
        out_shape=jax.ShapeDtypeStruct((N, D), jnp.float32),
    )(h, s0, s1, w1h_a, w1h_b, b1h, w2h, b2h, lnw, lnb)


def kernel(h, x, edge_index, W1m, b1m, W2m, b2m, W1h, b1h, W2h, b2h,
           ln_w, ln_b, centers, gamma):
    src = edge_index[0].astype(jnp.int32)
    dst = edge_index[1].astype(jnp.int32)
    x0 = x[:, 0]
    x1 = x[:, 1]
    cen_row = centers[None, :]
    gam_row = jnp.full((1, NRBF), gamma, jnp.float32)

    a_tab, b_tab = _make_tables(h, W1m[:D], W1m[D:2 * D])
    oa, ob, r2 = _sc_gather_fn()(a_tab, b_tab, src, dst, x0, x1)
    m = _make_edges(oa, ob, r2.reshape(E // 8, 8), W1m[2 * D:], b1m[None, :],
                    W2m, b2m[None, :], cen_row, gam_row)
    s_part = _sc_scatter_fn()(m, dst)
    out = _make_nodes(h, s_part[0, :N], s_part[1, :N], W1h[:D], W1h[D:],
                      b1h[None, :], W2h, b2h[None, :], ln_w[None, :],
                      ln_b[None, :])
    return out
